# baseline (speedup 1.0000x reference)
import functools

import jax
import jax.numpy as jnp
from jax import lax
from jax.experimental import pallas as pl
from jax.experimental.pallas import tpu as pltpu
from jax.experimental.pallas import tpu_sc as plsc

_NUM_LAYERS = 3
_T = 4096
_D = 512
_NC = 2
_NS = 16
_NW = _NC * _NS

_T_SC = 2304
_T_TC = _T - _T_SC
_ROWS = _T_SC // _NW
_CH = 72
_CPI = _ROWS // _CH
_NBUF = 3
_NV = _D // 16
_CHUNK_TC = 256
_NSTEPS_TC = _T_TC // _CHUNK_TC


def _sc_sums_body(p_now, a_now, b_now, s_now, p_pre, a_pre, b_pre, s_pre,
                  out_hbm, *scratch):
    bufs = scratch[:_NBUF]
    acc = scratch[_NBUF]
    sems = scratch[_NBUF + 1:]
    wid = lax.axis_index("c") * _NS + lax.axis_index("s")
    base = wid * _ROWS
    inputs = (p_now, a_now, b_now, s_now, p_pre, a_pre, b_pre, s_pre)
    nunits = 8 * _CPI

    def start(u):
        k, c = divmod(u, _CPI)
        return pltpu.async_copy(
            inputs[k].at[pl.ds(base + c * _CH, _CH), :, :], bufs[u % _NBUF],
            sems[u % _NBUF])

    def accumulate(bref, regs):
        @plsc.parallel_loop(0, _CH, carry=regs, unroll=4)
        def final(r, c):
            return tuple(c[v] + bref[r, 0, pl.ds(v * 16, 16)]
                         for v in range(_NV))
        return final

    copies = [start(u) for u in range(_NBUF)]
    for k in range(8):
        regs = tuple(jnp.zeros((16,), jnp.float32) for _ in range(_NV))
        for c in range(_CPI):
            u = k * _CPI + c
            copies[u % _NBUF].wait()
            regs = accumulate(bufs[u % _NBUF], regs)
            if u + _NBUF < nunits:
                copies[u % _NBUF] = start(u + _NBUF)
        for v in range(_NV):
            acc[k, pl.ds(v * 16, 16)] = regs[v]
    pltpu.sync_copy(acc, out_hbm.at[wid])


_sc_sums = functools.partial(
    pl.kernel,
    out_type=jax.ShapeDtypeStruct((_NW, 8, _D), jnp.float32),
    mesh=plsc.VectorSubcoreMesh(core_axis_name="c", subcore_axis_name="s"),
    scratch_types=(
        [pltpu.VMEM((_CH, 1, _D), jnp.float32)] * _NBUF +
        [pltpu.VMEM((8, _D), jnp.float32)] +
        [pltpu.SemaphoreType.DMA] * _NBUF
    ),
)(_sc_sums_body)


def _tc_sums_body(p_now, a_now, b_now, s_now, p_pre, a_pre, b_pre, s_pre,
                  out_ref, acc_ref):
    step = pl.program_id(0)

    @pl.when(step == 0)
    def _init():
        acc_ref[...] = jnp.zeros_like(acc_ref)

    refs = (p_now, a_now, b_now, s_now, p_pre, a_pre, b_pre, s_pre)
    sums = jnp.concatenate(
        [jnp.sum(r[...], axis=(0, 1)).reshape(1, _D) for r in refs], axis=0)
    acc_ref[...] += sums

    @pl.when(step == _NSTEPS_TC - 1)
    def _emit():
        out_ref[...] = acc_ref[...]


def _epi_body(sc_part, tc_part, W, wih, whh, bih, bhh, out_ref):
    x0 = (jnp.sum(sc_part[...], axis=0) + tc_part[...]) * (1.0 / _T)
    tgt = x0[4:8, :]
    row = jax.lax.broadcasted_iota(jnp.int32, (8, _D), 0)
    bih_v = bih[...]
    bhh_v = bhh[...]
    x = x0
    for i in range(_NUM_LAYERS):
        m = jnp.dot(x, W[i, :, :], preferred_element_type=jnp.float32)
        msum = jnp.sum(jnp.where(row >= 4, m, 0.0), axis=0, keepdims=True)
        agg = jnp.where(row < 4, msum, 0.0)
        gi = jax.lax.dot_general(agg, wih[...], (((1,), (1,)), ((), ())),
                                 preferred_element_type=jnp.float32) + bih_v
        gh = jax.lax.dot_general(x, whh[...], (((1,), (1,)), ((), ())),
                                 preferred_element_type=jnp.float32) + bhh_v
        r_g = jax.nn.sigmoid(gi[:, :_D] + gh[:, :_D])
        z_g = jax.nn.sigmoid(gi[:, _D:2 * _D] + gh[:, _D:2 * _D])
        n_g = jnp.tanh(gi[:, 2 * _D:] + r_g * gh[:, 2 * _D:])
        x = (1.0 - z_g) * n_g + z_g * x
    diff = x[0:4, :] - tgt
    d2 = jnp.sum(diff * diff, axis=1, keepdims=True)
    out_ref[...] = (10.0 / _D) * jnp.sum(d2, axis=0, keepdims=True)


def kernel(pair_now, person_1_now, person_2_now, scene_now,
           pair_pre, person_1_pre, person_2_pre, scene_pre,
           W, w_ih, w_hh, b_ih, b_hh):
    data = (pair_now, person_1_now, person_2_now, scene_now,
            pair_pre, person_1_pre, person_2_pre, scene_pre)
    bih2 = b_ih.reshape(1, 3 * _D)
    bhh2 = b_hh.reshape(1, 3 * _D)

    sc_partials = _sc_sums(*data)

    data_spec = pl.BlockSpec(
        (_CHUNK_TC, 1, _D), lambda i: (_T_SC // _CHUNK_TC + i, 0, 0))
    tc_partial = pl.pallas_call(
        _tc_sums_body,
        grid=(_NSTEPS_TC,),
        in_specs=[data_spec] * 8,
        out_specs=pl.BlockSpec((8, _D), lambda i: (0, 0)),
        out_shape=jax.ShapeDtypeStruct((8, _D), jnp.float32),
        scratch_shapes=[pltpu.VMEM((8, _D), jnp.float32)],
        compiler_params=pltpu.CompilerParams(
            dimension_semantics=("arbitrary",)),
    )(*data)

    full = lambda shape: pl.BlockSpec(shape, lambda _n=len(shape): (0,) * _n)
    out = pl.pallas_call(
        _epi_body,
        grid=(),
        in_specs=[
            full((_NW, 8, _D)),
            full((8, _D)),
            full((_NUM_LAYERS, _D, _D)),
            full((3 * _D, _D)),
            full((3 * _D, _D)),
            full((1, 3 * _D)),
            full((1, 3 * _D)),
        ],
        out_specs=full((1, 1)),
        out_shape=jax.ShapeDtypeStruct((1, 1), jnp.float32),
    )(sc_partials, tc_partial, W, w_ih, w_hh, bih2, bhh2)
    return out[0, 0]


# speedup vs baseline: 1.0077x; 1.0077x over previous
import functools

import jax
import jax.numpy as jnp
from jax import lax
from jax.experimental import pallas as pl
from jax.experimental.pallas import tpu as pltpu
from jax.experimental.pallas import tpu_sc as plsc

_NUM_LAYERS = 3
_T = 4096
_D = 512
_NC = 2
_NS = 16
_NW = _NC * _NS

_T_SC = 2560
_T_TC = _T - _T_SC
_ROWS = _T_SC // _NW
_CH = 40
_CPI = _ROWS // _CH
_NBUF = 4
_NV = _D // 16
_CHUNK_TC = 512
_NSTEPS_TC = _T_TC // _CHUNK_TC


def _sc_sums_body(p_now, a_now, b_now, s_now, p_pre, a_pre, b_pre, s_pre,
                  out_hbm, *scratch):
    bufs = scratch[:_NBUF]
    acc = scratch[_NBUF]
    sems = scratch[_NBUF + 1:]
    wid = lax.axis_index("c") * _NS + lax.axis_index("s")
    base = wid * _ROWS
    inputs = (p_now, a_now, b_now, s_now, p_pre, a_pre, b_pre, s_pre)
    nunits = 8 * _CPI

    def start(u):
        k, c = divmod(u, _CPI)
        return pltpu.async_copy(
            inputs[k].at[pl.ds(base + c * _CH, _CH), :, :], bufs[u % _NBUF],
            sems[u % _NBUF])

    def accumulate(bref, regs):
        @plsc.parallel_loop(0, _CH, carry=regs, unroll=4)
        def final(r, c):
            return tuple(c[v] + bref[r, 0, pl.ds(v * 16, 16)]
                         for v in range(_NV))
        return final

    copies = [start(u) for u in range(_NBUF)]
    for k in range(8):
        regs = tuple(jnp.zeros((16,), jnp.float32) for _ in range(_NV))
        for c in range(_CPI):
            u = k * _CPI + c
            copies[u % _NBUF].wait()
            regs = accumulate(bufs[u % _NBUF], regs)
            if u + _NBUF < nunits:
                copies[u % _NBUF] = start(u + _NBUF)
        for v in range(_NV):
            acc[k, pl.ds(v * 16, 16)] = regs[v]
    pltpu.sync_copy(acc, out_hbm.at[wid])


_sc_sums = functools.partial(
    pl.kernel,
    out_type=jax.ShapeDtypeStruct((_NW, 8, _D), jnp.float32),
    mesh=plsc.VectorSubcoreMesh(core_axis_name="c", subcore_axis_name="s"),
    scratch_types=(
        [pltpu.VMEM((_CH, 1, _D), jnp.float32)] * _NBUF +
        [pltpu.VMEM((8, _D), jnp.float32)] +
        [pltpu.SemaphoreType.DMA] * _NBUF
    ),
)(_sc_sums_body)


def _tc_sums_body(p_now, a_now, b_now, s_now, p_pre, a_pre, b_pre, s_pre,
                  out_ref, acc_ref):
    step = pl.program_id(0)

    @pl.when(step == 0)
    def _init():
        acc_ref[...] = jnp.zeros_like(acc_ref)

    refs = (p_now, a_now, b_now, s_now, p_pre, a_pre, b_pre, s_pre)
    sums = jnp.concatenate(
        [jnp.sum(r[...], axis=(0, 1)).reshape(1, _D) for r in refs], axis=0)
    acc_ref[...] += sums

    @pl.when(step == _NSTEPS_TC - 1)
    def _emit():
        out_ref[...] = acc_ref[...]


def _epi_body(sc_part, tc_part, W, wih, whh, bih, bhh, out_ref):
    x0 = (jnp.sum(sc_part[...], axis=0) + tc_part[...]) * (1.0 / _T)
    tgt = x0[4:8, :]
    row = jax.lax.broadcasted_iota(jnp.int32, (8, _D), 0)
    bih_v = bih[...]
    bhh_v = bhh[...]
    x = x0
    for i in range(_NUM_LAYERS):
        m = jnp.dot(x, W[i, :, :], preferred_element_type=jnp.float32)
        msum = jnp.sum(jnp.where(row >= 4, m, 0.0), axis=0, keepdims=True)
        agg = jnp.where(row < 4, msum, 0.0)
        gi = jax.lax.dot_general(agg, wih[...], (((1,), (1,)), ((), ())),
                                 preferred_element_type=jnp.float32) + bih_v
        gh = jax.lax.dot_general(x, whh[...], (((1,), (1,)), ((), ())),
                                 preferred_element_type=jnp.float32) + bhh_v
        r_g = jax.nn.sigmoid(gi[:, :_D] + gh[:, :_D])
        z_g = jax.nn.sigmoid(gi[:, _D:2 * _D] + gh[:, _D:2 * _D])
        n_g = jnp.tanh(gi[:, 2 * _D:] + r_g * gh[:, 2 * _D:])
        x = (1.0 - z_g) * n_g + z_g * x
    diff = x[0:4, :] - tgt
    d2 = jnp.sum(diff * diff, axis=1, keepdims=True)
    out_ref[...] = (10.0 / _D) * jnp.sum(d2, axis=0, keepdims=True)


def kernel(pair_now, person_1_now, person_2_now, scene_now,
           pair_pre, person_1_pre, person_2_pre, scene_pre,
           W, w_ih, w_hh, b_ih, b_hh):
    data = (pair_now, person_1_now, person_2_now, scene_now,
            pair_pre, person_1_pre, person_2_pre, scene_pre)
    bih2 = b_ih.reshape(1, 3 * _D)
    bhh2 = b_hh.reshape(1, 3 * _D)

    sc_partials = _sc_sums(*data)

    data_spec = pl.BlockSpec(
        (_CHUNK_TC, 1, _D), lambda i: (_T_SC // _CHUNK_TC + i, 0, 0))
    tc_partial = pl.pallas_call(
        _tc_sums_body,
        grid=(_NSTEPS_TC,),
        in_specs=[data_spec] * 8,
        out_specs=pl.BlockSpec((8, _D), lambda i: (0, 0)),
        out_shape=jax.ShapeDtypeStruct((8, _D), jnp.float32),
        scratch_shapes=[pltpu.VMEM((8, _D), jnp.float32)],
        compiler_params=pltpu.CompilerParams(
            dimension_semantics=("arbitrary",)),
    )(*data)

    full = lambda shape: pl.BlockSpec(shape, lambda _n=len(shape): (0,) * _n)
    out = pl.pallas_call(
        _epi_body,
        grid=(),
        in_specs=[
            full((_NW, 8, _D)),
            full((8, _D)),
            full((_NUM_LAYERS, _D, _D)),
            full((3 * _D, _D)),
            full((3 * _D, _D)),
            full((1, 3 * _D)),
            full((1, 3 * _D)),
        ],
        out_specs=full((1, 1)),
        out_shape=jax.ShapeDtypeStruct((1, 1), jnp.float32),
    )(sc_partials, tc_partial, W, w_ih, w_hh, bih2, bhh2)
    return out[0, 0]


# speedup vs baseline: 1.0182x; 1.0105x over previous
import functools

import jax
import jax.numpy as jnp
from jax import lax
from jax.experimental import pallas as pl
from jax.experimental.pallas import tpu as pltpu
from jax.experimental.pallas import tpu_sc as plsc

_NUM_LAYERS = 3
_T = 4096
_D = 512
_NC = 2
_NS = 16
_NW = _NC * _NS

_T_SC = 2560
_T_TC = _T - _T_SC
_ROWS = _T_SC // _NW
_CH = 80
_CPI = _ROWS // _CH
_NBUF = 3
_NV = _D // 16
_CHUNK_TC = 512
_NSTEPS_TC = _T_TC // _CHUNK_TC


def _sc_sums_body(p_now, a_now, b_now, s_now, p_pre, a_pre, b_pre, s_pre,
                  out_hbm, *scratch):
    bufs = scratch[:_NBUF]
    acc = scratch[_NBUF]
    sems = scratch[_NBUF + 1:]
    wid = lax.axis_index("c") * _NS + lax.axis_index("s")
    base = wid * _ROWS
    inputs = (p_now, a_now, b_now, s_now, p_pre, a_pre, b_pre, s_pre)
    nunits = 8 * _CPI

    def start(u):
        k, c = divmod(u, _CPI)
        return pltpu.async_copy(
            inputs[k].at[pl.ds(base + c * _CH, _CH), :, :], bufs[u % _NBUF],
            sems[u % _NBUF])

    def accumulate(bref, regs):
        @plsc.parallel_loop(0, _CH, carry=regs, unroll=4)
        def final(r, c):
            return tuple(c[v] + bref[r, 0, pl.ds(v * 16, 16)]
                         for v in range(_NV))
        return final

    copies = [start(u) for u in range(_NBUF)]
    for k in range(8):
        regs = tuple(jnp.zeros((16,), jnp.float32) for _ in range(_NV))
        for c in range(_CPI):
            u = k * _CPI + c
            copies[u % _NBUF].wait()
            regs = accumulate(bufs[u % _NBUF], regs)
            if u + _NBUF < nunits:
                copies[u % _NBUF] = start(u + _NBUF)
        for v in range(_NV):
            acc[k, pl.ds(v * 16, 16)] = regs[v]
    pltpu.sync_copy(acc, out_hbm.at[wid])


_sc_sums = functools.partial(
    pl.kernel,
    out_type=jax.ShapeDtypeStruct((_NW, 8, _D), jnp.float32),
    mesh=plsc.VectorSubcoreMesh(core_axis_name="c", subcore_axis_name="s"),
    scratch_types=(
        [pltpu.VMEM((_CH, 1, _D), jnp.float32)] * _NBUF +
        [pltpu.VMEM((8, _D), jnp.float32)] +
        [pltpu.SemaphoreType.DMA] * _NBUF
    ),
)(_sc_sums_body)


def _tc_sums_body(p_now, a_now, b_now, s_now, p_pre, a_pre, b_pre, s_pre,
                  out_ref, acc_ref):
    step = pl.program_id(0)

    @pl.when(step == 0)
    def _init():
        acc_ref[...] = jnp.zeros_like(acc_ref)

    refs = (p_now, a_now, b_now, s_now, p_pre, a_pre, b_pre, s_pre)
    sums = jnp.concatenate(
        [jnp.sum(r[...], axis=(0, 1)).reshape(1, _D) for r in refs], axis=0)
    acc_ref[...] += sums

    @pl.when(step == _NSTEPS_TC - 1)
    def _emit():
        out_ref[...] = acc_ref[...]


def _epi_body(sc_part, tc_part, W, wih, whh, bih, bhh, out_ref):
    x0 = (jnp.sum(sc_part[...], axis=0) + tc_part[...]) * (1.0 / _T)
    tgt = x0[4:8, :]
    row = jax.lax.broadcasted_iota(jnp.int32, (8, _D), 0)
    bih_v = bih[...]
    bhh_v = bhh[...]
    x = x0
    for i in range(_NUM_LAYERS):
        m = jnp.dot(x, W[i, :, :], preferred_element_type=jnp.float32)
        msum = jnp.sum(jnp.where(row >= 4, m, 0.0), axis=0, keepdims=True)
        agg = jnp.where(row < 4, msum, 0.0)
        gi = jax.lax.dot_general(agg, wih[...], (((1,), (1,)), ((), ())),
                                 preferred_element_type=jnp.float32) + bih_v
        gh = jax.lax.dot_general(x, whh[...], (((1,), (1,)), ((), ())),
                                 preferred_element_type=jnp.float32) + bhh_v
        r_g = jax.nn.sigmoid(gi[:, :_D] + gh[:, :_D])
        z_g = jax.nn.sigmoid(gi[:, _D:2 * _D] + gh[:, _D:2 * _D])
        n_g = jnp.tanh(gi[:, 2 * _D:] + r_g * gh[:, 2 * _D:])
        x = (1.0 - z_g) * n_g + z_g * x
    diff = x[0:4, :] - tgt
    d2 = jnp.sum(diff * diff, axis=1, keepdims=True)
    out_ref[...] = (10.0 / _D) * jnp.sum(d2, axis=0, keepdims=True)


def kernel(pair_now, person_1_now, person_2_now, scene_now,
           pair_pre, person_1_pre, person_2_pre, scene_pre,
           W, w_ih, w_hh, b_ih, b_hh):
    data = (pair_now, person_1_now, person_2_now, scene_now,
            pair_pre, person_1_pre, person_2_pre, scene_pre)
    bih2 = b_ih.reshape(1, 3 * _D)
    bhh2 = b_hh.reshape(1, 3 * _D)

    sc_partials = _sc_sums(*data)

    data_spec = pl.BlockSpec(
        (_CHUNK_TC, 1, _D), lambda i: (_T_SC // _CHUNK_TC + i, 0, 0))
    tc_partial = pl.pallas_call(
        _tc_sums_body,
        grid=(_NSTEPS_TC,),
        in_specs=[data_spec] * 8,
        out_specs=pl.BlockSpec((8, _D), lambda i: (0, 0)),
        out_shape=jax.ShapeDtypeStruct((8, _D), jnp.float32),
        scratch_shapes=[pltpu.VMEM((8, _D), jnp.float32)],
        compiler_params=pltpu.CompilerParams(
            dimension_semantics=("arbitrary",)),
    )(*data)

    full = lambda shape: pl.BlockSpec(shape, lambda _n=len(shape): (0,) * _n)
    out = pl.pallas_call(
        _epi_body,
        grid=(),
        in_specs=[
            full((_NW, 8, _D)),
            full((8, _D)),
            full((_NUM_LAYERS, _D, _D)),
            full((3 * _D, _D)),
            full((3 * _D, _D)),
            full((1, 3 * _D)),
            full((1, 3 * _D)),
        ],
        out_specs=full((1, 1)),
        out_shape=jax.ShapeDtypeStruct((1, 1), jnp.float32),
    )(sc_partials, tc_partial, W, w_ih, w_hh, bih2, bhh2)
    return out[0, 0]
